# baseline (speedup 1.0000x reference)
import functools

import jax
import jax.numpy as jnp
from jax.experimental import pallas as pl

N = 10000
E = 160000
DF = 128
DE = 16
G = 200

BN = 1000
BE = 2000


def _leaky(x):
    return jnp.where(x > 0, x, 0.01 * x)


def _elu(x):
    return jnp.where(x > 0, x, jnp.exp(jnp.minimum(x, 0.0)) - 1.0)


def _k1_body(nf_ref, wpn_ref, bpn_ref, w1a_ref, b1_ref, w2a_ref,
             hv_ref, proj_ref, dot_ref):
    nf = nf_ref[...]
    hv = _leaky(jnp.dot(nf, wpn_ref[...], preferred_element_type=jnp.float32)
                + bpn_ref[...])
    hv_ref[...] = hv
    proj_ref[...] = (jnp.dot(nf, w1a_ref[...], preferred_element_type=jnp.float32)
                     + b1_ref[...])
    dot_ref[...] = jnp.dot(hv, w2a_ref[...], preferred_element_type=jnp.float32)


def _node_ctx(nf, wpn, bpn, w1a, b1, w2a):
    grid = (N // BN,)
    return pl.pallas_call(
        _k1_body,
        grid=grid,
        in_specs=[
            pl.BlockSpec((BN, DF), lambda i: (i, 0)),
            pl.BlockSpec((DF, G), lambda i: (0, 0)),
            pl.BlockSpec((1, G), lambda i: (0, 0)),
            pl.BlockSpec((DF, G), lambda i: (0, 0)),
            pl.BlockSpec((1, G), lambda i: (0, 0)),
            pl.BlockSpec((G, 1), lambda i: (0, 0)),
        ],
        out_specs=[
            pl.BlockSpec((BN, G), lambda i: (i, 0)),
            pl.BlockSpec((BN, G), lambda i: (i, 0)),
            pl.BlockSpec((BN, 1), lambda i: (i, 0)),
        ],
        out_shape=[
            jax.ShapeDtypeStruct((N, G), jnp.float32),
            jax.ShapeDtypeStruct((N, G), jnp.float32),
            jax.ShapeDtypeStruct((N, 1), jnp.float32),
        ],
    )(nf, wpn, bpn, w1a, b1, w2a)


def _k2_body(psrc_ref, ef_ref, ddst_ref, w1b_ref, w2b_ref, b2_ref,
             wet_ref, bet_ref, het_ref, logit_ref):
    he1 = _leaky(psrc_ref[...]
                 + jnp.dot(ef_ref[...], w1b_ref[...],
                           preferred_element_type=jnp.float32))
    logit_ref[...] = _leaky(
        ddst_ref[...]
        + jnp.dot(he1, w2b_ref[...], preferred_element_type=jnp.float32)
        + b2_ref[...])
    het_ref[...] = (jnp.dot(he1, wet_ref[...], preferred_element_type=jnp.float32)
                    + bet_ref[...])


def _edge_ctx(psrc, ef, ddst, w1b, w2b, b2, wet, bet):
    grid = (E // BE,)
    return pl.pallas_call(
        _k2_body,
        grid=grid,
        in_specs=[
            pl.BlockSpec((BE, G), lambda i: (i, 0)),
            pl.BlockSpec((BE, DE), lambda i: (i, 0)),
            pl.BlockSpec((BE, 1), lambda i: (i, 0)),
            pl.BlockSpec((DE, G), lambda i: (0, 0)),
            pl.BlockSpec((G, 1), lambda i: (0, 0)),
            pl.BlockSpec((1, 1), lambda i: (0, 0)),
            pl.BlockSpec((G, G), lambda i: (0, 0)),
            pl.BlockSpec((1, G), lambda i: (0, 0)),
        ],
        out_specs=[
            pl.BlockSpec((BE, G), lambda i: (i, 0)),
            pl.BlockSpec((BE, 1), lambda i: (i, 0)),
        ],
        out_shape=[
            jax.ShapeDtypeStruct((E, G), jnp.float32),
            jax.ShapeDtypeStruct((E, 1), jnp.float32),
        ],
    )(psrc, ef, ddst, w1b, w2b, b2, wet, bet)


def _k3_body(x_ref, h_ref,
             wir_ref, wiz_ref, win_ref, bir_ref, biz_ref, bin_ref,
             whr_ref, whz_ref, whn_ref, bhr_ref, bhz_ref, bhn_ref,
             o_ref):
    x = _elu(x_ref[...])
    h = h_ref[...]

    def mm(a, w_ref, b_ref):
        return (jnp.dot(a, w_ref[...], preferred_element_type=jnp.float32)
                + b_ref[...])

    r = jax.nn.sigmoid(mm(x, wir_ref, bir_ref) + mm(h, whr_ref, bhr_ref))
    z = jax.nn.sigmoid(mm(x, wiz_ref, biz_ref) + mm(h, whz_ref, bhz_ref))
    nn_ = jnp.tanh(mm(x, win_ref, bin_ref) + r * mm(h, whn_ref, bhn_ref))
    out = (1.0 - z) * nn_ + z * h
    o_ref[...] = jnp.maximum(out, 0.0)


def _gru_relu(x_raw, h, gp):
    wi, bi, wh, bh = gp['Wi'], gp['bi'], gp['Wh'], gp['bh']
    ws = [wi[:, :G], wi[:, G:2 * G], wi[:, 2 * G:],
          bi[:G].reshape(1, G), bi[G:2 * G].reshape(1, G), bi[2 * G:].reshape(1, G),
          wh[:, :G], wh[:, G:2 * G], wh[:, 2 * G:],
          bh[:G].reshape(1, G), bh[G:2 * G].reshape(1, G), bh[2 * G:].reshape(1, G)]
    grid = (N // BN,)
    w_specs = ([pl.BlockSpec((G, G), lambda i: (0, 0))] * 3
               + [pl.BlockSpec((1, G), lambda i: (0, 0))] * 3) * 2
    return pl.pallas_call(
        _k3_body,
        grid=grid,
        in_specs=[pl.BlockSpec((BN, G), lambda i: (i, 0)),
                  pl.BlockSpec((BN, G), lambda i: (i, 0))] + w_specs,
        out_specs=pl.BlockSpec((BN, G), lambda i: (i, 0)),
        out_shape=jax.ShapeDtypeStruct((N, G), jnp.float32),
    )(x_raw, h, *ws)


def _k4_body(x_ref, w_ref, b_ref, wa_ref, wb_ref, o_ref, da_ref, db_ref):
    x = x_ref[...]
    o_ref[...] = (jnp.dot(x, w_ref[...], preferred_element_type=jnp.float32)
                  + b_ref[...])
    da_ref[...] = jnp.dot(x, wa_ref[...], preferred_element_type=jnp.float32)
    db_ref[...] = jnp.dot(x, wb_ref[...], preferred_element_type=jnp.float32)


def _node_proj(x, w, b, wa, wb):
    grid = (N // BN,)
    return pl.pallas_call(
        _k4_body,
        grid=grid,
        in_specs=[
            pl.BlockSpec((BN, G), lambda i: (i, 0)),
            pl.BlockSpec((G, G), lambda i: (0, 0)),
            pl.BlockSpec((1, G), lambda i: (0, 0)),
            pl.BlockSpec((G, 1), lambda i: (0, 0)),
            pl.BlockSpec((G, 1), lambda i: (0, 0)),
        ],
        out_specs=[
            pl.BlockSpec((BN, G), lambda i: (i, 0)),
            pl.BlockSpec((BN, 1), lambda i: (i, 0)),
            pl.BlockSpec((BN, 1), lambda i: (i, 0)),
        ],
        out_shape=[
            jax.ShapeDtypeStruct((N, G), jnp.float32),
            jax.ShapeDtypeStruct((N, 1), jnp.float32),
            jax.ShapeDtypeStruct((N, 1), jnp.float32),
        ],
    )(x, w, b, wa, wb)


def _edge_softmax(logits, dst, n):
    m = jax.ops.segment_max(logits, dst, num_segments=n)
    m = jnp.where(jnp.isfinite(m), m, 0.0)
    e = jnp.exp(logits - m[dst])
    s = jax.ops.segment_sum(e, dst, num_segments=n)
    return e / (s[dst] + 1e-9)


@jax.jit
def _impl(node_feats, edge_feats, edge_index, params):
    src = edge_index[0]
    dst = edge_index[1]
    c = params['ctx']

    hv_new, proj_n, hv_dot = _node_ctx(
        node_feats, c['Wpn'], c['bpn'].reshape(1, G),
        c['Wpe1'][:DF], c['bpe1'].reshape(1, G), c['Wpe2'][:G])
    het, logits = _edge_ctx(
        proj_n[src], edge_feats, hv_dot[dst],
        c['Wpe1'][DF:], c['Wpe2'][G:], c['bpe2'].reshape(1, 1),
        c['Wet'], c['bet'].reshape(1, G))
    a = _edge_softmax(logits[:, 0], dst, N)
    cagg = jax.ops.segment_sum(a[:, None] * het, dst, num_segments=N)
    h = _gru_relu(cagg, hv_new, c['gru'])

    for l in params['gnn']:
        hp, hd, hs = _node_proj(h, l['Wpn'], l['bpn'].reshape(1, G),
                                l['Wpe'][:G], l['Wpe'][G:])
        lg = _leaky(hd[dst, 0] + hs[src, 0] + l['bpe'][0])
        a = _edge_softmax(lg, dst, N)
        cagg = jax.ops.segment_sum(a[:, None] * hp[src], dst, num_segments=N)
        h = _gru_relu(cagg, h, l['gru'])

    def gru_small(x, hstate, gp):
        gi = x @ gp['Wi'] + gp['bi']
        gh = hstate @ gp['Wh'] + gp['bh']
        ir, iz, inn = jnp.split(gi, 3, axis=-1)
        hr, hz, hn = jnp.split(gh, 3, axis=-1)
        r = jax.nn.sigmoid(ir + hr)
        z = jax.nn.sigmoid(iz + hz)
        nn_ = jnp.tanh(inn + r * hn)
        return (1.0 - z) * nn_ + z * hstate

    g_feats = jnp.sum(h, axis=0, keepdims=True)
    for r in params['ro']:
        hv, hdot, _ = _node_proj(h, r['Wpn'], r['bpn'].reshape(1, G),
                                 r['Wcl'][G:], r['Wcl'][G:])
        gdot = jnp.maximum(g_feats, 0.0) @ r['Wcl'][:G]
        z = _leaky(hdot + gdot + r['bcl'])
        aw = jax.nn.softmax(z, axis=0)
        g_repr = jnp.sum(aw * hv, axis=0, keepdims=True)
        g_feats = jnp.maximum(gru_small(_elu(g_repr), g_feats, r['gru']), 0.0)

    return g_feats @ params['Wout'] + params['bout']


def kernel(node_feats, edge_feats, edge_index, params):
    return _impl(node_feats, edge_feats, edge_index, params)


# speedup vs baseline: 1.0013x; 1.0013x over previous
import functools

import jax
import jax.numpy as jnp
from jax.experimental import pallas as pl

N = 10000
E = 160000
DF = 128
DE = 16
G = 200

BN = 2000
BE = 4000


def _leaky(x):
    return jnp.where(x > 0, x, 0.01 * x)


def _elu(x):
    return jnp.where(x > 0, x, jnp.exp(jnp.minimum(x, 0.0)) - 1.0)


def _k1_body(nf_ref, wpn_ref, bpn_ref, w1a_ref, b1_ref, w2a_ref,
             hv_ref, proj_ref, dot_ref):
    nf = nf_ref[...]
    hv = _leaky(jnp.dot(nf, wpn_ref[...], preferred_element_type=jnp.float32)
                + bpn_ref[...])
    hv_ref[...] = hv
    proj_ref[...] = (jnp.dot(nf, w1a_ref[...], preferred_element_type=jnp.float32)
                     + b1_ref[...])
    dot_ref[...] = jnp.dot(hv, w2a_ref[...], preferred_element_type=jnp.float32)


def _node_ctx(nf, wpn, bpn, w1a, b1, w2a):
    grid = (N // BN,)
    return pl.pallas_call(
        _k1_body,
        grid=grid,
        in_specs=[
            pl.BlockSpec((BN, DF), lambda i: (i, 0)),
            pl.BlockSpec((DF, G), lambda i: (0, 0)),
            pl.BlockSpec((1, G), lambda i: (0, 0)),
            pl.BlockSpec((DF, G), lambda i: (0, 0)),
            pl.BlockSpec((1, G), lambda i: (0, 0)),
            pl.BlockSpec((G, 1), lambda i: (0, 0)),
        ],
        out_specs=[
            pl.BlockSpec((BN, G), lambda i: (i, 0)),
            pl.BlockSpec((BN, G), lambda i: (i, 0)),
            pl.BlockSpec((BN, 1), lambda i: (i, 0)),
        ],
        out_shape=[
            jax.ShapeDtypeStruct((N, G), jnp.float32),
            jax.ShapeDtypeStruct((N, G), jnp.float32),
            jax.ShapeDtypeStruct((N, 1), jnp.float32),
        ],
    )(nf, wpn, bpn, w1a, b1, w2a)


def _k2_body(psrc_ref, ef_ref, ddst_ref, w1b_ref, w2b_ref, b2_ref,
             wet_ref, bet_ref, het_ref, logit_ref):
    he1 = _leaky(psrc_ref[...]
                 + jnp.dot(ef_ref[...], w1b_ref[...],
                           preferred_element_type=jnp.float32))
    logit_ref[...] = _leaky(
        ddst_ref[...]
        + jnp.dot(he1, w2b_ref[...], preferred_element_type=jnp.float32)
        + b2_ref[...])
    het_ref[...] = (jnp.dot(he1, wet_ref[...], preferred_element_type=jnp.float32)
                    + bet_ref[...])


def _edge_ctx(psrc, ef, ddst, w1b, w2b, b2, wet, bet):
    grid = (E // BE,)
    return pl.pallas_call(
        _k2_body,
        grid=grid,
        in_specs=[
            pl.BlockSpec((BE, G), lambda i: (i, 0)),
            pl.BlockSpec((BE, DE), lambda i: (i, 0)),
            pl.BlockSpec((BE, 1), lambda i: (i, 0)),
            pl.BlockSpec((DE, G), lambda i: (0, 0)),
            pl.BlockSpec((G, 1), lambda i: (0, 0)),
            pl.BlockSpec((1, 1), lambda i: (0, 0)),
            pl.BlockSpec((G, G), lambda i: (0, 0)),
            pl.BlockSpec((1, G), lambda i: (0, 0)),
        ],
        out_specs=[
            pl.BlockSpec((BE, G), lambda i: (i, 0)),
            pl.BlockSpec((BE, 1), lambda i: (i, 0)),
        ],
        out_shape=[
            jax.ShapeDtypeStruct((E, G), jnp.float32),
            jax.ShapeDtypeStruct((E, 1), jnp.float32),
        ],
    )(psrc, ef, ddst, w1b, w2b, b2, wet, bet)


def _k3_body(x_ref, h_ref,
             wir_ref, wiz_ref, win_ref, bir_ref, biz_ref, bin_ref,
             whr_ref, whz_ref, whn_ref, bhr_ref, bhz_ref, bhn_ref,
             o_ref):
    x = _elu(x_ref[...])
    h = h_ref[...]

    def mm(a, w_ref, b_ref):
        return (jnp.dot(a, w_ref[...], preferred_element_type=jnp.float32)
                + b_ref[...])

    r = jax.nn.sigmoid(mm(x, wir_ref, bir_ref) + mm(h, whr_ref, bhr_ref))
    z = jax.nn.sigmoid(mm(x, wiz_ref, biz_ref) + mm(h, whz_ref, bhz_ref))
    nn_ = jnp.tanh(mm(x, win_ref, bin_ref) + r * mm(h, whn_ref, bhn_ref))
    out = (1.0 - z) * nn_ + z * h
    o_ref[...] = jnp.maximum(out, 0.0)


def _gru_relu(x_raw, h, gp):
    wi, bi, wh, bh = gp['Wi'], gp['bi'], gp['Wh'], gp['bh']
    ws = [wi[:, :G], wi[:, G:2 * G], wi[:, 2 * G:],
          bi[:G].reshape(1, G), bi[G:2 * G].reshape(1, G), bi[2 * G:].reshape(1, G),
          wh[:, :G], wh[:, G:2 * G], wh[:, 2 * G:],
          bh[:G].reshape(1, G), bh[G:2 * G].reshape(1, G), bh[2 * G:].reshape(1, G)]
    grid = (N // BN,)
    w_specs = ([pl.BlockSpec((G, G), lambda i: (0, 0))] * 3
               + [pl.BlockSpec((1, G), lambda i: (0, 0))] * 3) * 2
    return pl.pallas_call(
        _k3_body,
        grid=grid,
        in_specs=[pl.BlockSpec((BN, G), lambda i: (i, 0)),
                  pl.BlockSpec((BN, G), lambda i: (i, 0))] + w_specs,
        out_specs=pl.BlockSpec((BN, G), lambda i: (i, 0)),
        out_shape=jax.ShapeDtypeStruct((N, G), jnp.float32),
    )(x_raw, h, *ws)


def _k4_body(x_ref, w_ref, b_ref, wa_ref, wb_ref, o_ref, da_ref, db_ref):
    x = x_ref[...]
    o_ref[...] = (jnp.dot(x, w_ref[...], preferred_element_type=jnp.float32)
                  + b_ref[...])
    da_ref[...] = jnp.dot(x, wa_ref[...], preferred_element_type=jnp.float32)
    db_ref[...] = jnp.dot(x, wb_ref[...], preferred_element_type=jnp.float32)


def _node_proj(x, w, b, wa, wb):
    grid = (N // BN,)
    return pl.pallas_call(
        _k4_body,
        grid=grid,
        in_specs=[
            pl.BlockSpec((BN, G), lambda i: (i, 0)),
            pl.BlockSpec((G, G), lambda i: (0, 0)),
            pl.BlockSpec((1, G), lambda i: (0, 0)),
            pl.BlockSpec((G, 1), lambda i: (0, 0)),
            pl.BlockSpec((G, 1), lambda i: (0, 0)),
        ],
        out_specs=[
            pl.BlockSpec((BN, G), lambda i: (i, 0)),
            pl.BlockSpec((BN, 1), lambda i: (i, 0)),
            pl.BlockSpec((BN, 1), lambda i: (i, 0)),
        ],
        out_shape=[
            jax.ShapeDtypeStruct((N, G), jnp.float32),
            jax.ShapeDtypeStruct((N, 1), jnp.float32),
            jax.ShapeDtypeStruct((N, 1), jnp.float32),
        ],
    )(x, w, b, wa, wb)


def _edge_softmax(logits, dst, n):
    m = jax.ops.segment_max(logits, dst, num_segments=n)
    m = jnp.where(jnp.isfinite(m), m, 0.0)
    e = jnp.exp(logits - m[dst])
    s = jax.ops.segment_sum(e, dst, num_segments=n)
    return e / (s[dst] + 1e-9)


@jax.jit
def _impl(node_feats, edge_feats, edge_index, params):
    src = edge_index[0]
    dst = edge_index[1]
    c = params['ctx']

    hv_new, proj_n, hv_dot = _node_ctx(
        node_feats, c['Wpn'], c['bpn'].reshape(1, G),
        c['Wpe1'][:DF], c['bpe1'].reshape(1, G), c['Wpe2'][:G])
    het, logits = _edge_ctx(
        proj_n[src], edge_feats, hv_dot[dst],
        c['Wpe1'][DF:], c['Wpe2'][G:], c['bpe2'].reshape(1, 1),
        c['Wet'], c['bet'].reshape(1, G))
    a = _edge_softmax(logits[:, 0], dst, N)
    cagg = jax.ops.segment_sum(a[:, None] * het, dst, num_segments=N)
    h = _gru_relu(cagg, hv_new, c['gru'])

    for l in params['gnn']:
        hp, hd, hs = _node_proj(h, l['Wpn'], l['bpn'].reshape(1, G),
                                l['Wpe'][:G], l['Wpe'][G:])
        lg = _leaky(hd[dst, 0] + hs[src, 0] + l['bpe'][0])
        a = _edge_softmax(lg, dst, N)
        cagg = jax.ops.segment_sum(a[:, None] * hp[src], dst, num_segments=N)
        h = _gru_relu(cagg, h, l['gru'])

    def gru_small(x, hstate, gp):
        gi = x @ gp['Wi'] + gp['bi']
        gh = hstate @ gp['Wh'] + gp['bh']
        ir, iz, inn = jnp.split(gi, 3, axis=-1)
        hr, hz, hn = jnp.split(gh, 3, axis=-1)
        r = jax.nn.sigmoid(ir + hr)
        z = jax.nn.sigmoid(iz + hz)
        nn_ = jnp.tanh(inn + r * hn)
        return (1.0 - z) * nn_ + z * hstate

    g_feats = jnp.sum(h, axis=0, keepdims=True)
    for r in params['ro']:
        hv, hdot, _ = _node_proj(h, r['Wpn'], r['bpn'].reshape(1, G),
                                 r['Wcl'][G:], r['Wcl'][G:])
        gdot = jnp.maximum(g_feats, 0.0) @ r['Wcl'][:G]
        z = _leaky(hdot + gdot + r['bcl'])
        aw = jax.nn.softmax(z, axis=0)
        g_repr = jnp.sum(aw * hv, axis=0, keepdims=True)
        g_feats = jnp.maximum(gru_small(_elu(g_repr), g_feats, r['gru']), 0.0)

    return g_feats @ params['Wout'] + params['bout']


def kernel(node_feats, edge_feats, edge_index, params):
    return _impl(node_feats, edge_feats, edge_index, params)
